# baseline (speedup 1.0000x reference)
import jax
import jax.numpy as jnp
from jax.experimental import pallas as pl

B = 4096
N = 50
D = 128
H1 = 256
OUT = 128
R = 3

BB = 64
NP = 64
T = BB * NP


def _lk(x):
    return jnp.maximum(x, x * jnp.asarray(0.01, x.dtype))


def _dot_t(x, w, out_dtype):
    return jax.lax.dot_general(x, w, (((1,), (1,)), ((), ())),
                               preferred_element_type=out_dtype)


MC = 128
KE = R * H1 + MC


def _fused_body(u_ref, i_ref, x_ref, s_ref,
                uw0_ref, ub0_ref, uw1_ref, ub1_ref,
                aw0_ref, ab0_ref, w1e_ref,
                pred_ref):
    f32 = jnp.float32
    bf16 = jnp.bfloat16

    u = u_ref[...].astype(bf16)
    i = i_ref[...].astype(bf16)
    h_ui = _lk(
        _dot_t(u, uw0_ref[:, :D], f32)
        + _dot_t(i, uw0_ref[:, D:], f32)
        + ub0_ref[...]
    )
    ue = _lk(_dot_t(h_ui.astype(bf16), uw1_ref[...], f32) + ub1_ref[...])
    ue_b = ue[:, None, :]

    zpad = jnp.zeros((BB, NP - N, 2 * D), dtype=bf16)
    x = jnp.concatenate([x_ref[...], zpad], axis=1).reshape(T, 2 * D)

    h_all = _dot_t(x, aw0_ref[...], f32) + ab0_ref[...]
    h_all = _lk(h_all)

    s = s_ref[...]
    sp = jnp.concatenate(
        [s, jnp.zeros((BB, NP - N), jnp.int32)], axis=1)[:, :, None]
    h3 = h_all.reshape(BB, NP, R * H1)
    lane = jax.lax.broadcasted_iota(jnp.int32, (1, 1, R * H1), 2) // H1
    hm = jnp.where(lane == sp, h3, 0.0)
    lanec = jax.lax.broadcasted_iota(jnp.int32, (1, 1, MC), 2)
    mcols = jnp.where(lanec == sp, 1.0, 0.0)
    hx = jnp.concatenate([hm, mcols], axis=2).reshape(T, KE).astype(bf16)

    out = _lk(_dot_t(hx, w1e_ref[...], f32))
    pred = jnp.sum(out.reshape(BB, NP, OUT) * ue_b, axis=-1)
    pred_ref[...] = pred[:, :N]


def kernel(u_emb, i_emb, a_emb, o_emb, s, ui_W0, ui_b0, ui_W1, ui_b1,
           ao_W0, ao_b0, ao_W1, ao_b1):
    bf16 = jnp.bfloat16
    x_all = jnp.concatenate([a_emb, o_emb], axis=-1).astype(bf16)
    uw0 = ui_W0.astype(bf16)
    uw1 = ui_W1.astype(bf16)
    aw0 = ao_W0.reshape(R * H1, 2 * D).astype(bf16)
    ab0 = ao_b0.reshape(R * H1)
    w1cat = jnp.concatenate([ao_W1[r] for r in range(R)], axis=1)
    bcols = jnp.zeros((OUT, MC), ao_b1.dtype).at[:, :R].set(ao_b1.T)
    w1e = jnp.concatenate([w1cat, bcols], axis=1).astype(bf16)
    s32 = s.astype(jnp.int32)

    grid = (B // BB,)

    def const(shape):
        nd = len(shape)
        return pl.BlockSpec(shape, lambda i: (0,) * nd)

    out = pl.pallas_call(
        _fused_body,
        grid=grid,
        in_specs=[
            pl.BlockSpec((BB, D), lambda i: (i, 0)),
            pl.BlockSpec((BB, D), lambda i: (i, 0)),
            pl.BlockSpec((BB, N, 2 * D), lambda i: (i, 0, 0)),
            pl.BlockSpec((BB, N), lambda i: (i, 0)),
            const((H1, 2 * D)), const((H1,)),
            const((OUT, H1)), const((OUT,)),
            const((R * H1, 2 * D)), const((R * H1,)),
            const((OUT, KE)),
        ],
        out_specs=pl.BlockSpec((BB, N), lambda i: (i, 0)),
        out_shape=jax.ShapeDtypeStruct((B, N), jnp.float32),
    )(u_emb, i_emb, x_all, s32,
      uw0, ui_b0, uw1, ui_b1,
      aw0, ab0, w1e)
    return out


# speedup vs baseline: 1.3249x; 1.3249x over previous
import jax
import jax.numpy as jnp
from jax.experimental import pallas as pl

B = 4096
N = 50
D = 128
H1 = 256
OUT = 128
R = 3

BB = 64
NP = 64
T = BB * NP


def _lk(x):
    return jnp.maximum(x, x * jnp.asarray(0.01, x.dtype))


def _dot_t(x, w, out_dtype):
    return jax.lax.dot_general(x, w, (((1,), (1,)), ((), ())),
                               preferred_element_type=out_dtype)


def _fused_body(u_ref, i_ref, a_ref, o_ref, s_ref,
                uw0_ref, ub0_ref, uw1_ref, ub1_ref,
                aw0_ref, ab0_ref, aw1_ref, ab1_ref,
                pred_ref):
    f32 = jnp.float32
    bf16 = jnp.bfloat16

    u = u_ref[...].astype(bf16)
    i = i_ref[...].astype(bf16)
    h_ui = _lk(
        _dot_t(u, uw0_ref[:, :D], f32)
        + _dot_t(i, uw0_ref[:, D:], f32)
        + ub0_ref[...]
    )
    ue = _lk(_dot_t(h_ui.astype(bf16), uw1_ref[...], f32) + ub1_ref[...])
    ue_b = ue[:, None, :]

    xao = jnp.concatenate([a_ref[...].astype(bf16),
                           o_ref[...].astype(bf16)], axis=2)
    zpad = jnp.zeros((BB, NP - N, 2 * D), dtype=bf16)
    x = jnp.concatenate([xao, zpad], axis=1).reshape(T, 2 * D)

    h_all = _dot_t(x, aw0_ref[...], f32) + ab0_ref[...]
    h_all = _lk(h_all.astype(bf16))

    s = s_ref[...]
    pred = jnp.zeros((BB, N), dtype=f32)
    for r in range(R):
        h_r = h_all[:, r * H1:(r + 1) * H1]
        out_r = _lk(_dot_t(h_r, aw1_ref[r], f32) + ab1_ref[r])
        p_r = jnp.sum(out_r.reshape(BB, NP, OUT) * ue_b, axis=-1)
        pred = pred + jnp.where(s == r, p_r[:, :N], 0.0)
    pred_ref[...] = pred


def kernel(u_emb, i_emb, a_emb, o_emb, s, ui_W0, ui_b0, ui_W1, ui_b1,
           ao_W0, ao_b0, ao_W1, ao_b1):
    bf16 = jnp.bfloat16
    uw0 = ui_W0.astype(bf16)
    uw1 = ui_W1.astype(bf16)
    aw0 = ao_W0.reshape(R * H1, 2 * D).astype(bf16)
    ab0 = ao_b0.reshape(R * H1)
    aw1 = ao_W1.astype(bf16)
    s32 = s.astype(jnp.int32)

    grid = (B // BB,)

    def const(shape):
        nd = len(shape)
        return pl.BlockSpec(shape, lambda i: (0,) * nd)

    out = pl.pallas_call(
        _fused_body,
        grid=grid,
        in_specs=[
            pl.BlockSpec((BB, D), lambda i: (i, 0)),
            pl.BlockSpec((BB, D), lambda i: (i, 0)),
            pl.BlockSpec((BB, N, D), lambda i: (i, 0, 0)),
            pl.BlockSpec((BB, N, D), lambda i: (i, 0, 0)),
            pl.BlockSpec((BB, N), lambda i: (i, 0)),
            const((H1, 2 * D)), const((H1,)),
            const((OUT, H1)), const((OUT,)),
            const((R * H1, 2 * D)), const((R * H1,)),
            const((R, OUT, H1)), const((R, OUT)),
        ],
        out_specs=pl.BlockSpec((BB, N), lambda i: (i, 0)),
        out_shape=jax.ShapeDtypeStruct((B, N), jnp.float32),
    )(u_emb, i_emb, a_emb, o_emb, s32,
      uw0, ui_b0, uw1, ui_b1,
      aw0, ab0, aw1, ao_b1)
    return out
